# Initial kernel scaffold; baseline (speedup 1.0000x reference)
import jax
import jax.numpy as jnp
from jax.experimental import pallas as pl


def kernel(sequence, time_index_sequence, variable_index_sequence, sector_index_sequence, var_table, sect_table, t2v_w, t2v_b):
    raise NotImplementedError("write your pallas kernel here")


# speedup vs baseline: 3.6542x; 3.6542x over previous
import jax
import jax.numpy as jnp
from jax import lax
from jax.experimental import pallas as pl
from jax.experimental.pallas import tpu as pltpu
from jax.experimental.pallas import tpu_sc as plsc

B = 4096
S = 200
F = 32
E_VAR = 32
E_TIME = 16
E_SECT = 32
E_OUT = F + E_VAR + E_TIME + E_SECT
N = B * S
NC, NS = 2, 16
NW = NC * NS
RW = N // NW
C = 512
NCHUNK = RW // C
GW = 64
GSUB = C // GW


def _t2v_body(t_ref, w_ref, b_ref, out_ref):
    xa = t_ref[...] * w_ref[...] + b_ref[...]
    lane = lax.broadcasted_iota(jnp.int32, (2, E_TIME), 1)
    val = jnp.where(lane == 0, xa, jnp.sin(xa))
    row = lax.broadcasted_iota(jnp.int32, (C, E_TIME), 0)
    out_ref[...] = jnp.where(
        row % 2 == 0,
        jnp.broadcast_to(val[0:1, :], (C, E_TIME)),
        jnp.broadcast_to(val[1:2, :], (C, E_TIME)),
    )


def _time_pattern(t2, t2v_w, t2v_b):
    return pl.pallas_call(
        _t2v_body,
        out_shape=jax.ShapeDtypeStruct((C, E_TIME), jnp.float32),
    )(t2.reshape(2, 1), t2v_w, t2v_b)


def _sc_body(seq_hbm, vidx_hbm, sidx_hbm, var_hbm, sect_hbm, pat_hbm, out_hbm,
             vidx_v, sidx_v, seq_v, var_v, sect_v, pat_v, sem_v, sem_s):
    cid = lax.axis_index("c")
    sid = lax.axis_index("s")
    wid = sid * NC + cid
    base0 = wid * RW
    pltpu.sync_copy(pat_hbm, pat_v)

    def chunk(g, carry):
        base = pl.multiple_of(base0 + g * C, C)
        brow = pl.multiple_of(base // GW, GSUB)
        pltpu.sync_copy(vidx_hbm.at[pl.ds(brow, GSUB)], vidx_v)
        pltpu.sync_copy(sidx_hbm.at[pl.ds(brow, GSUB)], sidx_v)
        cps = []
        for j in range(GSUB):
            cps.append(pltpu.async_copy(
                var_hbm.at[vidx_v.at[j]], var_v.at[pl.ds(j * GW, GW)], sem_v))
            cps.append(pltpu.async_copy(
                sect_hbm.at[sidx_v.at[j]], sect_v.at[pl.ds(j * GW, GW)], sem_s))
        pltpu.sync_copy(seq_hbm.at[pl.ds(base, C)], seq_v)
        pltpu.sync_copy(seq_v, out_hbm.at[pl.ds(base, C), pl.ds(0, F)])
        pltpu.sync_copy(pat_v, out_hbm.at[pl.ds(base, C), pl.ds(F + E_VAR, E_TIME)])
        for cp in cps:
            cp.wait()
        pltpu.sync_copy(var_v, out_hbm.at[pl.ds(base, C), pl.ds(F, E_VAR)])
        pltpu.sync_copy(sect_v, out_hbm.at[pl.ds(base, C), pl.ds(F + E_VAR + E_TIME, E_SECT)])
        return carry

    lax.fori_loop(0, NCHUNK, chunk, 0)


_sc_call = pl.kernel(
    _sc_body,
    out_type=jax.ShapeDtypeStruct((N, E_OUT), jnp.float32),
    mesh=plsc.VectorSubcoreMesh(
        core_axis_name="c", subcore_axis_name="s",
        num_cores=NC, num_subcores=NS),
    scratch_types=[
        pltpu.VMEM((GSUB, GW), jnp.int32),
        pltpu.VMEM((GSUB, GW), jnp.int32),
        pltpu.VMEM((C, F), jnp.float32),
        pltpu.VMEM((C, E_VAR), jnp.float32),
        pltpu.VMEM((C, E_SECT), jnp.float32),
        pltpu.VMEM((C, E_TIME), jnp.float32),
        pltpu.SemaphoreType.DMA,
        pltpu.SemaphoreType.DMA,
    ],
    compiler_params=pltpu.CompilerParams(use_tc_tiling_on_sc=False),
)


def kernel(sequence, time_index_sequence, variable_index_sequence,
           sector_index_sequence, var_table, sect_table, t2v_w, t2v_b):
    t2 = time_index_sequence[0, :2].astype(jnp.float32)
    pattern = _time_pattern(t2, t2v_w, t2v_b)
    seqf = sequence.reshape(N, F)
    vidx = variable_index_sequence.reshape(N // GW, GW)
    sidx = sector_index_sequence.reshape(N // GW, GW)
    outf = _sc_call(seqf, vidx, sidx, var_table, sect_table, pattern)
    return outf.reshape(B, S, E_OUT)
